# Initial kernel scaffold; baseline (speedup 1.0000x reference)
import jax
import jax.numpy as jnp
from jax.experimental import pallas as pl


def kernel(tags, keypoints):
    raise NotImplementedError("write your pallas kernel here")


# speedup vs baseline: 76.1784x; 76.1784x over previous
import functools

import jax
import jax.numpy as jnp
import numpy as np
from jax import lax
from jax.experimental import pallas as pl
from jax.experimental.pallas import tpu as pltpu
from jax.experimental.pallas import tpu_sc as plsc

B, L = 16, 65536
P, J = 30, 17
PJ = 32
NPAD = P * PJ

_I = np.arange(64)
_K = np.arange(32)
_RECIP_TAB = np.concatenate([
    1.0 / np.maximum(_I, 1),
    1.0 / np.maximum(_K * (_K - 1) / 2.0, 1.0),
]).astype(np.float32)


def _aeloss_body(tags_hbm, idx_hbm, w_hbm, tab_hbm, out_hbm,
                 tags_v, idx_v, w_v, tab_v, means_v, out_v):
    wid = lax.axis_index("s") * 2 + lax.axis_index("c")

    @pl.when(wid < B)
    def _():
        b = wid
        pltpu.sync_copy(tags_hbm.at[b], tags_v)
        pltpu.sync_copy(idx_hbm.at[b], idx_v)
        pltpu.sync_copy(w_hbm.at[b], w_v)
        pltpu.sync_copy(tab_hbm, tab_v)

        iota = lax.iota(jnp.int32, 16)
        push_acc = jnp.zeros((16,), jnp.float32)
        cur = jnp.int32(0)
        ma = jnp.zeros((16,), jnp.float32)
        mb = jnp.zeros((16,), jnp.float32)

        for p in range(P):
            i0 = idx_v[pl.ds(PJ * p, 16)]
            i1 = idx_v[pl.ds(PJ * p + 16, 16)]
            w0 = w_v[pl.ds(PJ * p, 16)]
            w1 = w_v[pl.ds(PJ * p + 16, 16)]
            t0 = plsc.load_gather(tags_v, [i0])
            t1 = plsc.load_gather(tags_v, [i1])
            n = jnp.sum(w0 + w1)
            s = jnp.sum(t0 * w0 + t1 * w1)
            has = n > 0.5
            ni = n.astype(jnp.int32)
            rn = plsc.load_gather(tab_v, [jnp.full((16,), 0, jnp.int32) + ni])
            m = s * rn
            d0 = t0 - m
            d1 = t1 - m
            var = jnp.sum(d0 * d0 * w0 + d1 * d1 * w1) * rn
            m = jnp.where(has, m, 0.0)
            push_acc = push_acc + jnp.where(has, var, 0.0)
            cur = cur + jnp.where(has, 1, 0).astype(jnp.int32)
            if p < 16:
                ma = jnp.where(iota == p, m, ma)
            else:
                mb = jnp.where(iota == (p - 16), m, mb)

        means_v[pl.ds(0, 16)] = ma
        means_v[pl.ds(16, 16)] = mb

        kf = cur.astype(jnp.float32)
        mask_a = jnp.where(iota < cur, 1.0, 0.0)
        mask_b = jnp.where((iota + 16) < cur, 1.0, 0.0)
        acc = jnp.zeros((16,), jnp.float32)
        for j in range(P):
            bj = plsc.load_gather(means_v, [jnp.full((16,), j, jnp.int32)])
            da = ma - bj
            db = mb - bj
            e = jnp.exp(-(da * da)) * mask_a + jnp.exp(-(db * db)) * mask_b
            acc = acc + e * jnp.where(cur > j, 1.0, 0.0)
        total = jnp.sum(acc)

        rk = plsc.load_gather(tab_v, [jnp.full((16,), 0, jnp.int32) + cur])
        rp = plsc.load_gather(tab_v, [jnp.full((16,), 64, jnp.int32) + cur])
        pull = (total - kf) * 0.5
        pull = jnp.where(cur > 1, pull * rp, jnp.zeros((16,)) + pull)
        pull = pull * 0.5
        push = jnp.where(cur > 0, push_acc * rk, push_acc)

        out_v[...] = jnp.where(iota == 0, pull,
                               jnp.where(iota == 1, push, 0.0))
        pltpu.sync_copy(out_v, out_hbm.at[b])


_aeloss = functools.partial(
    pl.kernel,
    out_type=jax.ShapeDtypeStruct((B, 16), jnp.float32),
    mesh=plsc.VectorSubcoreMesh(core_axis_name="c", subcore_axis_name="s"),
    compiler_params=pltpu.CompilerParams(needs_layout_passes=False),
    scratch_types=[
        pltpu.VMEM((L,), jnp.float32),
        pltpu.VMEM((NPAD,), jnp.int32),
        pltpu.VMEM((NPAD,), jnp.float32),
        pltpu.VMEM((96,), jnp.float32),
        pltpu.VMEM((32,), jnp.float32),
        pltpu.VMEM((16,), jnp.float32),
    ],
)(_aeloss_body)


@jax.jit
def kernel(tags, keypoints):
    tags2 = tags.reshape(B, L)
    idx = keypoints[..., 0]
    flg = keypoints[..., 1].astype(jnp.float32)
    idxp = jnp.zeros((B, P, PJ), jnp.int32).at[:, :, :J].set(idx)
    wp = jnp.zeros((B, P, PJ), jnp.float32).at[:, :, :J].set(flg)
    out = _aeloss(tags2, idxp.reshape(B, NPAD), wp.reshape(B, NPAD),
                  jnp.asarray(_RECIP_TAB))
    return out[:, :2]


# speedup vs baseline: 98.0278x; 1.2868x over previous
import functools

import jax
import jax.numpy as jnp
import numpy as np
from jax import lax
from jax.experimental import pallas as pl
from jax.experimental.pallas import tpu as pltpu
from jax.experimental.pallas import tpu_sc as plsc

B, L = 16, 65536
LROWS = L // 16
P, J = 30, 17
PP = 32
NT = J * PP
NTP = 640
NCHUNK = 5

_I = np.arange(64)
_K = np.arange(32)
_RECIP_TAB = np.concatenate([
    1.0 / np.maximum(_I, 1),
    1.0 / np.maximum(_K * (_K - 1) / 2.0, 1.0),
]).astype(np.float32)


def _aeloss_body(tags_hbm, rows_hbm, lanes_hbm, w_hbm, tab_hbm, out_hbm,
                 rows_v, lanes_v, w_v, gath_v, tab_v, means_v, out_v, sem):
    wid = lax.axis_index("s") * 2 + lax.axis_index("c")

    @pl.when(wid < B)
    def _():
        b = wid
        pltpu.sync_copy(rows_hbm.at[b], rows_v)
        tab = tags_hbm.at[b]
        copies = [
            pltpu.async_copy(
                tab.at[rows_v.at[pl.ds(c * 128, 128)]],
                gath_v.at[pl.ds(c * 128, 128)],
                sem,
            )
            for c in range(NCHUNK)
        ]
        pltpu.sync_copy(lanes_hbm.at[b], lanes_v)
        pltpu.sync_copy(w_hbm.at[b], w_v)
        pltpu.sync_copy(tab_hbm, tab_v)
        for cp in copies:
            cp.wait()

        iota = lax.iota(jnp.int32, 16)
        na = jnp.zeros((16,), jnp.float32)
        nb = jnp.zeros((16,), jnp.float32)
        sa = jnp.zeros((16,), jnp.float32)
        sb = jnp.zeros((16,), jnp.float32)
        qa = jnp.zeros((16,), jnp.float32)
        qb = jnp.zeros((16,), jnp.float32)
        for j in range(J):
            ta = plsc.load_gather(
                gath_v, [j * PP + iota, lanes_v[pl.ds(j * PP, 16)]])
            tb = plsc.load_gather(
                gath_v, [j * PP + 16 + iota, lanes_v[pl.ds(j * PP + 16, 16)]])
            wa = w_v[pl.ds(j * PP, 16)]
            wb = w_v[pl.ds(j * PP + 16, 16)]
            twa = ta * wa
            twb = tb * wb
            na = na + wa
            nb = nb + wb
            sa = sa + twa
            sb = sb + twb
            qa = qa + ta * twa
            qb = qb + tb * twb

        nia = na.astype(jnp.int32)
        nib = nb.astype(jnp.int32)
        rna = plsc.load_gather(tab_v, [nia])
        rnb = plsc.load_gather(tab_v, [nib])
        ma = sa * rna
        mb = sb * rnb
        va = qa - 2.0 * ma * sa + ma * ma * na
        vb = qb - 2.0 * mb * sb + mb * mb * nb

        has_a = nia > 0
        has_b = nib > 0
        zero = jnp.zeros((16,), jnp.float32)
        ma = jnp.where(has_a, ma, zero)
        mb = jnp.where(has_b, mb, zero)
        pushv = jnp.where(has_a, va * rna, zero) + jnp.where(has_b, vb * rnb, zero)
        push_acc = jnp.sum(pushv)
        cur = (plsc.all_reduce_population_count(has_a)
               + plsc.all_reduce_population_count(has_b))

        means_v[pl.ds(0, 16)] = ma
        means_v[pl.ds(16, 16)] = mb

        kf = cur.astype(jnp.float32)
        mask_a = jnp.where(iota < cur, 1.0, 0.0)
        mask_b = jnp.where((iota + 16) < cur, 1.0, 0.0)
        acc = jnp.zeros((16,), jnp.float32)
        for j in range(P):
            bj = plsc.load_gather(means_v, [jnp.full((16,), j, jnp.int32)])
            da = ma - bj
            db = mb - bj
            e = jnp.exp(-(da * da)) * mask_a + jnp.exp(-(db * db)) * mask_b
            acc = acc + e * jnp.where(cur > j, 1.0, 0.0)
        total = jnp.sum(acc)

        rk = plsc.load_gather(tab_v, [cur])
        rp = plsc.load_gather(tab_v, [cur + 64])
        pull = (total - kf) * 0.5
        pull = jnp.where(cur > 1, pull * rp, zero + pull)
        pull = pull * 0.5
        push = jnp.where(cur > 0, push_acc * rk, zero + push_acc)

        out_v[...] = jnp.where(iota == 0, pull,
                               jnp.where(iota == 1, push, 0.0))
        pltpu.sync_copy(out_v, out_hbm.at[b])


_aeloss = functools.partial(
    pl.kernel,
    out_type=jax.ShapeDtypeStruct((B, 16), jnp.float32),
    mesh=plsc.VectorSubcoreMesh(core_axis_name="c", subcore_axis_name="s"),
    compiler_params=pltpu.CompilerParams(
        needs_layout_passes=False, use_tc_tiling_on_sc=False),
    scratch_types=[
        pltpu.VMEM((NTP,), jnp.int32),
        pltpu.VMEM((NT,), jnp.int32),
        pltpu.VMEM((NT,), jnp.float32),
        pltpu.VMEM((NTP, 16), jnp.float32),
        pltpu.VMEM((96,), jnp.float32),
        pltpu.VMEM((32,), jnp.float32),
        pltpu.VMEM((16,), jnp.float32),
        pltpu.SemaphoreType.DMA,
    ],
)(_aeloss_body)


@jax.jit
def kernel(tags, keypoints):
    tags3 = tags.reshape(B, LROWS, 16)
    idx = keypoints[..., 0]
    flg = keypoints[..., 1].astype(jnp.float32)
    idx_t = jnp.zeros((B, J, PP), jnp.int32).at[:, :, :P].set(
        jnp.swapaxes(idx, 1, 2)).reshape(B, NT)
    w_t = jnp.zeros((B, J, PP), jnp.float32).at[:, :, :P].set(
        jnp.swapaxes(flg, 1, 2)).reshape(B, NT)
    rows = jnp.zeros((B, NTP), jnp.int32).at[:, :NT].set(idx_t >> 4)
    lanes = idx_t & 15
    out = _aeloss(tags3, rows, lanes, w_t, jnp.asarray(_RECIP_TAB))
    return out[:, :2]


# speedup vs baseline: 106.2765x; 1.0841x over previous
import functools

import jax
import jax.numpy as jnp
import numpy as np
from jax import lax
from jax.experimental import pallas as pl
from jax.experimental.pallas import tpu as pltpu
from jax.experimental.pallas import tpu_sc as plsc

B, L = 16, 65536
LROWS = L // 16
P, J = 30, 17
PP = 32
NT = J * PP
KPW = 1088
CHUNKS = [(0, 128, 4), (128, 128, 8), (256, 128, 12), (384, 128, 16),
          (512, 32, 17)]

_I = np.arange(64)
_K = np.arange(32)
_RECIP_TAB = np.concatenate([
    1.0 / np.maximum(_I, 1),
    1.0 / np.maximum(_K * (_K - 1) / 2.0, 1.0),
]).astype(np.float32)


def _aeloss_body(tags_hbm, kp_hbm, tab_hbm, out_hbm,
                 kp_v, rows_v, lanes_v, w_v, gath_v, tab_v, means_v,
                 out_v, asm_v, out2_v, shared_v, sem):
    c = lax.axis_index("c")
    s = lax.axis_index("s")
    iota = lax.iota(jnp.int32, 16)

    @pl.when(s < 8)
    def _():
        b = c * 8 + s
        pltpu.sync_copy(kp_hbm.at[b], kp_v)
        pltpu.sync_copy(tab_hbm, tab_v)

        base_a = 34 * iota
        base_b = 34 * (iota + 16)
        copies = []
        tab3 = tags_hbm.at[b]
        ci = 0
        for j in range(J):
            ia = plsc.load_gather(kp_v, [base_a + 2 * j])
            ib = plsc.load_gather(kp_v, [base_b + 2 * j])
            fa = plsc.load_gather(kp_v, [base_a + 2 * j + 1])
            fb = plsc.load_gather(kp_v, [base_b + 2 * j + 1])
            rows_v[pl.ds(j * PP, 16)] = jnp.right_shift(ia, 4)
            rows_v[pl.ds(j * PP + 16, 16)] = jnp.right_shift(ib, 4)
            lanes_v[pl.ds(j * PP, 16)] = jnp.bitwise_and(ia, 15)
            lanes_v[pl.ds(j * PP + 16, 16)] = jnp.bitwise_and(ib, 15)
            w_v[pl.ds(j * PP, 16)] = fa.astype(jnp.float32)
            w_v[pl.ds(j * PP + 16, 16)] = fb.astype(jnp.float32)
            while ci < len(CHUNKS) and CHUNKS[ci][2] == j + 1:
                start, nsl, _ = CHUNKS[ci]
                copies.append(pltpu.async_copy(
                    tab3.at[rows_v.at[pl.ds(start, nsl)]],
                    gath_v.at[pl.ds(start, nsl)], sem))
                ci += 1

        na = jnp.zeros((16,), jnp.float32)
        nb = jnp.zeros((16,), jnp.float32)
        sa = jnp.zeros((16,), jnp.float32)
        sb = jnp.zeros((16,), jnp.float32)
        qa = jnp.zeros((16,), jnp.float32)
        qb = jnp.zeros((16,), jnp.float32)
        ci = 0
        for j in range(J):
            while ci < len(CHUNKS) and CHUNKS[ci][0] <= j * PP:
                copies[ci].wait()
                ci += 1
            ta = plsc.load_gather(
                gath_v, [j * PP + iota, lanes_v[pl.ds(j * PP, 16)]])
            tb = plsc.load_gather(
                gath_v, [j * PP + 16 + iota, lanes_v[pl.ds(j * PP + 16, 16)]])
            wa = w_v[pl.ds(j * PP, 16)]
            wb = w_v[pl.ds(j * PP + 16, 16)]
            twa = ta * wa
            twb = tb * wb
            na = na + wa
            nb = nb + wb
            sa = sa + twa
            sb = sb + twb
            qa = qa + ta * twa
            qb = qb + tb * twb
        for cp in copies[ci:]:
            cp.wait()

        nia = na.astype(jnp.int32)
        nib = nb.astype(jnp.int32)
        rna = plsc.load_gather(tab_v, [nia])
        rnb = plsc.load_gather(tab_v, [nib])
        ma = sa * rna
        mb = sb * rnb
        va = qa - 2.0 * ma * sa + ma * ma * na
        vb = qb - 2.0 * mb * sb + mb * mb * nb

        has_a = nia > 0
        has_b = nib > 0
        zero = jnp.zeros((16,), jnp.float32)
        ma = jnp.where(has_a, ma, zero)
        mb = jnp.where(has_b, mb, zero)
        pushv = jnp.where(has_a, va * rna, zero) + jnp.where(has_b, vb * rnb, zero)
        push_acc = jnp.sum(pushv)
        cur = (plsc.all_reduce_population_count(has_a)
               + plsc.all_reduce_population_count(has_b))

        means_v[pl.ds(0, 16)] = ma
        means_v[pl.ds(16, 16)] = mb

        kf = cur.astype(jnp.float32)
        mask_a = jnp.where(iota < cur, 1.0, 0.0)
        mask_b = jnp.where((iota + 16) < cur, 1.0, 0.0)
        acc = jnp.zeros((16,), jnp.float32)
        for j in range(P):
            bj = plsc.load_gather(means_v, [jnp.full((16,), j, jnp.int32)])
            da = ma - bj
            db = mb - bj
            e = jnp.exp(-(da * da)) * mask_a + jnp.exp(-(db * db)) * mask_b
            acc = acc + e * jnp.where(cur > j, 1.0, 0.0)
        total = jnp.sum(acc)

        rk = plsc.load_gather(tab_v, [cur])
        rp = plsc.load_gather(tab_v, [cur + 64])
        pull = (total - kf) * 0.5
        pull = jnp.where(cur > 1, pull * rp, zero + pull)
        pull = pull * 0.5
        push = jnp.where(cur > 0, push_acc * rk, zero + push_acc)

        out_v[...] = jnp.where(iota == 0, pull,
                               jnp.where(iota == 1, push, 0.0))
        pltpu.sync_copy(out_v, shared_v.at[s])

    plsc.subcore_barrier()

    @pl.when(s == 0)
    def _():
        pltpu.sync_copy(shared_v, asm_v)
        r = plsc.load_gather(
            asm_v, [jnp.right_shift(iota, 1), jnp.bitwise_and(iota, 1)])
        out2_v[...] = r
        pltpu.sync_copy(out2_v, out_hbm.at[pl.ds(c * 16, 16)])


_aeloss = functools.partial(
    pl.kernel,
    out_type=jax.ShapeDtypeStruct((2 * B,), jnp.float32),
    mesh=plsc.VectorSubcoreMesh(core_axis_name="c", subcore_axis_name="s"),
    compiler_params=pltpu.CompilerParams(
        needs_layout_passes=False, use_tc_tiling_on_sc=False),
    scratch_types=[
        pltpu.VMEM((KPW,), jnp.int32),
        pltpu.VMEM((NT,), jnp.int32),
        pltpu.VMEM((NT,), jnp.int32),
        pltpu.VMEM((NT,), jnp.float32),
        pltpu.VMEM((NT, 16), jnp.float32),
        pltpu.VMEM((96,), jnp.float32),
        pltpu.VMEM((32,), jnp.float32),
        pltpu.VMEM((16,), jnp.float32),
        pltpu.VMEM((8, 16), jnp.float32),
        pltpu.VMEM((16,), jnp.float32),
        pltpu.VMEM_SHARED((8, 16), jnp.float32),
        pltpu.SemaphoreType.DMA,
    ],
)(_aeloss_body)


@jax.jit
def kernel(tags, keypoints):
    tags3 = tags.reshape(B, LROWS, 16)
    kp = jnp.pad(keypoints.reshape(B, P * J * 2), ((0, 0), (0, KPW - P * J * 2)))
    out = _aeloss(tags3, kp, jnp.asarray(_RECIP_TAB))
    return out.reshape(B, 2)
